# baseline (speedup 1.0000x reference)
import functools

import jax
import jax.numpy as jnp
from jax import lax
from jax.experimental import pallas as pl
from jax.experimental.pallas import tpu as pltpu
from jax.experimental.pallas import tpu_sc as plsc

N_USERS = 1_000_000
N_ITEMS = 100_000
F = 16
B_U = 16384
B_I = 4096
W = 128

_NC = 2
_NS = 16
_NW = _NC * _NS

_CHUNK = 128
_U_PER_W = B_U // _NW
_I_PER_W = B_I // _NW
_U_CHUNKS = _U_PER_W // _CHUNK
_I_CHUNKS = _I_PER_W // _CHUNK


def _sc_gather(urows, irows, uf8, if8):
  mesh = plsc.VectorSubcoreMesh(core_axis_name="c", subcore_axis_name="s")

  @functools.partial(
      pl.kernel,
      out_type=[
          jax.ShapeDtypeStruct((B_U, W), jnp.float32),
          jax.ShapeDtypeStruct((B_I, W), jnp.float32),
      ],
      mesh=mesh,
      scratch_types=[
          pltpu.VMEM((_U_CHUNKS, _CHUNK), jnp.int32),
          pltpu.VMEM((_I_CHUNKS, _CHUNK), jnp.int32),
          pltpu.VMEM((_U_PER_W, W), jnp.float32),
          pltpu.VMEM((_I_PER_W, W), jnp.float32),
          pltpu.SemaphoreType.DMA,
      ],
  )
  def k(urows_hbm, irows_hbm, uf_hbm, if_hbm, u8_out, v8_out,
        idx_u, idx_i, rows_u, rows_i, sem):
    wid = lax.axis_index("s") * _NC + lax.axis_index("c")
    base_u = wid * _U_PER_W
    base_i = wid * _I_PER_W

    for c in range(_U_CHUNKS):
      pltpu.sync_copy(urows_hbm.at[pl.ds(base_u + c * _CHUNK, _CHUNK)],
                      idx_u.at[c])
    for c in range(_I_CHUNKS):
      pltpu.sync_copy(irows_hbm.at[pl.ds(base_i + c * _CHUNK, _CHUNK)],
                      idx_i.at[c])

    descs = []
    for c in range(_U_CHUNKS):
      descs.append(pltpu.async_copy(
          uf_hbm.at[idx_u.at[c]],
          rows_u.at[pl.ds(c * _CHUNK, _CHUNK)], sem))
    for c in range(_I_CHUNKS):
      descs.append(pltpu.async_copy(
          if_hbm.at[idx_i.at[c]],
          rows_i.at[pl.ds(c * _CHUNK, _CHUNK)], sem))
    for d in descs:
      d.wait()

    pltpu.sync_copy(rows_u, u8_out.at[pl.ds(base_u, _U_PER_W)])
    pltpu.sync_copy(rows_i, v8_out.at[pl.ds(base_i, _I_PER_W)])

  return k(urows, irows, uf8, if8)


_BU_BLK = 512
_NBUF = 4
_NSTEPS = B_U // _BU_BLK


def _make_mm_body(nsteps):
 def _mm_body(users_ref, items_ref, u8_ref, v8_ref, o_hbm, ring, sems):
  i = pl.program_id(0)
  buf = lax.rem(i, _NBUF)

  @pl.when(i >= _NBUF)
  def _():
    pltpu.make_async_copy(
        ring.at[buf],
        o_hbm.at[pl.ds((i - _NBUF) * _BU_BLK, _BU_BLK)],
        sems.at[buf]).wait()

  slot_u = users_ref[...] & 7
  slot_i = items_ref[...] & 7
  cu = lax.broadcasted_iota(jnp.int32, (_BU_BLK, W), 1) // F
  ci = lax.broadcasted_iota(jnp.int32, (B_I, W), 1) // F
  u8m = jnp.where(cu == slot_u, u8_ref[...], 0.0)
  v8m = jnp.where(ci == slot_i, v8_ref[...], 0.0)
  gc = lax.broadcasted_iota(jnp.int32, (W, W), 0) % F
  gd = lax.broadcasted_iota(jnp.int32, (W, W), 1) % F
  g = (gc == gd).astype(jnp.float32)
  w = jnp.dot(u8m, g, preferred_element_type=jnp.float32)
  ring.at[buf][...] = lax.dot_general(
      w, v8m,
      dimension_numbers=(((1,), (1,)), ((), ())),
      preferred_element_type=jnp.float32)

  pltpu.make_async_copy(
      ring.at[buf],
      o_hbm.at[pl.ds(i * _BU_BLK, _BU_BLK)],
      sems.at[buf]).start()

  @pl.when(i == nsteps - 1)
  def _():
    for j in range(_NBUF):
      pltpu.make_async_copy(
          ring.at[j],
          o_hbm.at[pl.ds((nsteps - _NBUF + j) * _BU_BLK, _BU_BLK)],
          sems.at[j]).wait()
 return _mm_body


def _tc_matmul(users, items, u8, v8, n_rows=B_U):
  nsteps = n_rows // _BU_BLK
  return pl.pallas_call(
      _make_mm_body(nsteps),
      grid=(nsteps,),
      in_specs=[
          pl.BlockSpec((_BU_BLK, 1), lambda i: (i, 0)),
          pl.BlockSpec((B_I, 1), lambda i: (0, 0)),
          pl.BlockSpec((_BU_BLK, W), lambda i: (i, 0)),
          pl.BlockSpec((B_I, W), lambda i: (0, 0)),
      ],
      out_specs=pl.BlockSpec(memory_space=pl.ANY),
      out_shape=jax.ShapeDtypeStruct((n_rows, B_I), jnp.float32),
      scratch_shapes=[
          pltpu.VMEM((_NBUF, _BU_BLK, B_I), jnp.float32),
          pltpu.SemaphoreType.DMA((_NBUF,)),
      ],
      compiler_params=pltpu.CompilerParams(
          dimension_semantics=("arbitrary",)),
  )(users.reshape(-1, 1), items.reshape(B_I, 1), u8, v8)


def kernel(users, items, user_factors, item_factors):
  users = users.astype(jnp.int32)
  items = items.astype(jnp.int32)
  uf8 = user_factors.reshape(N_USERS // 8, W)
  if8 = item_factors.reshape(N_ITEMS // 8, W)
  u8, v8 = _sc_gather(users >> 3, items >> 3, uf8, if8)
  top = _tc_matmul(users[:12288], items, u8[:12288], v8, 12288)
  bot = _tc_matmul(users[12288:], items, u8[12288:], v8, B_U - 12288)
  return jnp.concatenate([top, bot], axis=0)


# speedup vs baseline: 1.8230x; 1.8230x over previous
import functools

import jax
import jax.numpy as jnp
from jax import lax
from jax.experimental import pallas as pl
from jax.experimental.pallas import tpu as pltpu
from jax.experimental.pallas import tpu_sc as plsc

N_USERS = 1_000_000
N_ITEMS = 100_000
F = 16
B_U = 16384
B_I = 4096

_NC = 2
_NS = 16
_NW = _NC * _NS

_U_PER_W = B_U // _NW
_I_PER_W = B_I // _NW
_L = 16


def _gather_rows(table_hbm, idx_ref, rows_ref, sem, n_rows):
  iota = lax.iota(jnp.int32, _L)

  def group(g, _):
    vec = idx_ref[pl.ds(g * _L, _L)]
    descs = []
    for l in range(_L):
      r = lax.reduce_max(jnp.where(iota == l, vec, 0), axes=(0,))
      descs.append(pltpu.async_copy(
          table_hbm.at[pl.ds(r, 1)],
          rows_ref.at[pl.ds(g * _L + l, 1)], sem))
    for d in descs:
      d.wait()
    return 0

  lax.fori_loop(0, n_rows // _L, group, 0)


def _sc_gather(users, items, user_factors, item_factors):
  mesh = plsc.VectorSubcoreMesh(core_axis_name="c", subcore_axis_name="s")

  @functools.partial(
      pl.kernel,
      out_type=[
          jax.ShapeDtypeStruct((B_U, F), jnp.float32),
          jax.ShapeDtypeStruct((B_I, F), jnp.float32),
      ],
      mesh=mesh,
      scratch_types=[
          pltpu.VMEM((_U_PER_W,), jnp.int32),
          pltpu.VMEM((_I_PER_W,), jnp.int32),
          pltpu.VMEM((_U_PER_W, F), jnp.float32),
          pltpu.VMEM((_I_PER_W, F), jnp.float32),
          pltpu.SemaphoreType.DMA,
      ],
      compiler_params=pltpu.CompilerParams(needs_layout_passes=False),
  )
  def k(users_hbm, items_hbm, uf_hbm, if_hbm, u_out, v_out,
        idx_u, idx_i, rows_u, rows_i, sem):
    wid = lax.axis_index("s") * _NC + lax.axis_index("c")
    base_u = wid * _U_PER_W
    base_i = wid * _I_PER_W

    pltpu.sync_copy(users_hbm.at[pl.ds(base_u, _U_PER_W)], idx_u)
    pltpu.sync_copy(items_hbm.at[pl.ds(base_i, _I_PER_W)], idx_i)

    _gather_rows(uf_hbm, idx_u, rows_u, sem, _U_PER_W)
    _gather_rows(if_hbm, idx_i, rows_i, sem, _I_PER_W)

    pltpu.sync_copy(rows_u, u_out.at[pl.ds(base_u, _U_PER_W)])
    pltpu.sync_copy(rows_i, v_out.at[pl.ds(base_i, _I_PER_W)])

  return k(users, items, user_factors, item_factors)


_BU_BLK = 1024


def _mm_body(u_ref, v_ref, o_ref):
  o_ref[...] = lax.dot_general(
      u_ref[...], v_ref[...],
      dimension_numbers=(((1,), (1,)), ((), ())),
      preferred_element_type=jnp.float32)


def _tc_matmul(u, v):
  return pl.pallas_call(
      _mm_body,
      grid=(B_U // _BU_BLK,),
      in_specs=[
          pl.BlockSpec((_BU_BLK, F), lambda i: (i, 0)),
          pl.BlockSpec((B_I, F), lambda i: (0, 0)),
      ],
      out_specs=pl.BlockSpec((_BU_BLK, B_I), lambda i: (i, 0)),
      out_shape=jax.ShapeDtypeStruct((B_U, B_I), jnp.float32),
      compiler_params=pltpu.CompilerParams(
          dimension_semantics=("arbitrary",)),
  )(u, v)


def kernel(users, items, user_factors, item_factors):
  users = users.astype(jnp.int32)
  items = items.astype(jnp.int32)
  u, v = _sc_gather(users, items, user_factors, item_factors)
  return _tc_matmul(u, v)


# speedup vs baseline: 1.9362x; 1.0621x over previous
import functools

import jax
import jax.numpy as jnp
from jax import lax
from jax.experimental import pallas as pl
from jax.experimental.pallas import tpu as pltpu
from jax.experimental.pallas import tpu_sc as plsc

N_USERS = 1_000_000
N_ITEMS = 100_000
F = 16
B_U = 16384
B_I = 4096

_NC = 2
_NS = 16
_NW = _NC * _NS

_U_PER_W = B_U // _NW
_I_PER_W = B_I // _NW
_L = 16


def _gather_rows(table_hbm, idx_ref, rows_ref, sem, n_rows):
  iota = lax.iota(jnp.int32, _L)

  def group(g, _):
    vec = idx_ref[pl.ds(g * _L, _L)]
    for l in range(_L):
      r = lax.reduce_max(jnp.where(iota == l, vec, 0), axes=(0,))
      pltpu.async_copy(
          table_hbm.at[pl.ds(r, 1)],
          rows_ref.at[pl.ds(g * _L + l, 1)], sem)
    return 0

  lax.fori_loop(0, n_rows // _L, group, 0)


def _drain(table_hbm, rows_ref, sem, n_rows):
  pltpu.make_async_copy(
      table_hbm.at[pl.ds(0, n_rows)], rows_ref, sem).wait()


def _sc_gather(users, items, user_factors, item_factors):
  mesh = plsc.VectorSubcoreMesh(core_axis_name="c", subcore_axis_name="s")

  @functools.partial(
      pl.kernel,
      out_type=[
          jax.ShapeDtypeStruct((B_U, F), jnp.float32),
          jax.ShapeDtypeStruct((B_I, F), jnp.float32),
      ],
      mesh=mesh,
      scratch_types=[
          pltpu.VMEM((_U_PER_W,), jnp.int32),
          pltpu.VMEM((_I_PER_W,), jnp.int32),
          pltpu.VMEM((_U_PER_W, F), jnp.float32),
          pltpu.VMEM((_I_PER_W, F), jnp.float32),
          pltpu.SemaphoreType.DMA,
      ],
      compiler_params=pltpu.CompilerParams(needs_layout_passes=False),
  )
  def k(users_hbm, items_hbm, uf_hbm, if_hbm, u_out, v_out,
        idx_u, idx_i, rows_u, rows_i, sem):
    wid = lax.axis_index("s") * _NC + lax.axis_index("c")
    base_u = wid * _U_PER_W
    base_i = wid * _I_PER_W

    pltpu.sync_copy(users_hbm.at[pl.ds(base_u, _U_PER_W)], idx_u)
    pltpu.sync_copy(items_hbm.at[pl.ds(base_i, _I_PER_W)], idx_i)

    _gather_rows(uf_hbm, idx_u, rows_u, sem, _U_PER_W)
    _gather_rows(if_hbm, idx_i, rows_i, sem, _I_PER_W)
    _drain(uf_hbm, rows_u, sem, _U_PER_W)
    _drain(if_hbm, rows_i, sem, _I_PER_W)

    pltpu.sync_copy(rows_u, u_out.at[pl.ds(base_u, _U_PER_W)])
    pltpu.sync_copy(rows_i, v_out.at[pl.ds(base_i, _I_PER_W)])

  return k(users, items, user_factors, item_factors)


_BU_BLK = 1024


def _mm_body(u_ref, v_ref, o_ref):
  o_ref[...] = lax.dot_general(
      u_ref[...], v_ref[...],
      dimension_numbers=(((1,), (1,)), ((), ())),
      preferred_element_type=jnp.float32)


def _tc_matmul(u, v):
  return pl.pallas_call(
      _mm_body,
      grid=(B_U // _BU_BLK,),
      in_specs=[
          pl.BlockSpec((_BU_BLK, F), lambda i: (i, 0)),
          pl.BlockSpec((B_I, F), lambda i: (0, 0)),
      ],
      out_specs=pl.BlockSpec((_BU_BLK, B_I), lambda i: (i, 0)),
      out_shape=jax.ShapeDtypeStruct((B_U, B_I), jnp.float32),
      compiler_params=pltpu.CompilerParams(
          dimension_semantics=("arbitrary",)),
  )(u, v)


def kernel(users, items, user_factors, item_factors):
  users = users.astype(jnp.int32)
  items = items.astype(jnp.int32)
  u, v = _sc_gather(users, items, user_factors, item_factors)
  return _tc_matmul(u, v)


# speedup vs baseline: 1.9399x; 1.0019x over previous
import functools

import jax
import jax.numpy as jnp
from jax import lax
from jax.experimental import pallas as pl
from jax.experimental.pallas import tpu as pltpu
from jax.experimental.pallas import tpu_sc as plsc

N_USERS = 1_000_000
N_ITEMS = 100_000
F = 16
B_U = 16384
B_I = 4096

_NC = 2
_NS = 16
_NW = _NC * _NS

_U_PER_W = B_U // _NW
_I_PER_W = B_I // _NW
_L = 16


def _gather_rows(table_hbm, idx_ref, rows_ref, sem, n_rows):
  def group(g, _):
    vec = idx_ref[pl.ds(g * _L, _L)]
    for l in range(_L):
      r = vec[l]
      pltpu.async_copy(
          table_hbm.at[pl.ds(r, 1)],
          rows_ref.at[pl.ds(g * _L + l, 1)], sem)
    return 0

  lax.fori_loop(0, n_rows // _L, group, 0)


def _drain(table_hbm, rows_ref, sem, n_rows):
  pltpu.make_async_copy(
      table_hbm.at[pl.ds(0, n_rows)], rows_ref, sem).wait()


def _sc_gather(users, items, user_factors, item_factors):
  mesh = plsc.VectorSubcoreMesh(core_axis_name="c", subcore_axis_name="s")

  @functools.partial(
      pl.kernel,
      out_type=[
          jax.ShapeDtypeStruct((B_U, F), jnp.float32),
          jax.ShapeDtypeStruct((B_I, F), jnp.float32),
      ],
      mesh=mesh,
      scratch_types=[
          pltpu.VMEM((_U_PER_W,), jnp.int32),
          pltpu.VMEM((_I_PER_W,), jnp.int32),
          pltpu.VMEM((_U_PER_W, F), jnp.float32),
          pltpu.VMEM((_I_PER_W, F), jnp.float32),
          pltpu.SemaphoreType.DMA,
      ],
      compiler_params=pltpu.CompilerParams(needs_layout_passes=False),
  )
  def k(users_hbm, items_hbm, uf_hbm, if_hbm, u_out, v_out,
        idx_u, idx_i, rows_u, rows_i, sem):
    wid = lax.axis_index("s") * _NC + lax.axis_index("c")
    base_u = wid * _U_PER_W
    base_i = wid * _I_PER_W

    pltpu.sync_copy(users_hbm.at[pl.ds(base_u, _U_PER_W)], idx_u)
    pltpu.sync_copy(items_hbm.at[pl.ds(base_i, _I_PER_W)], idx_i)

    _gather_rows(uf_hbm, idx_u, rows_u, sem, _U_PER_W)
    _gather_rows(if_hbm, idx_i, rows_i, sem, _I_PER_W)
    _drain(uf_hbm, rows_u, sem, _U_PER_W)
    _drain(if_hbm, rows_i, sem, _I_PER_W)

    pltpu.sync_copy(rows_u, u_out.at[pl.ds(base_u, _U_PER_W)])
    pltpu.sync_copy(rows_i, v_out.at[pl.ds(base_i, _I_PER_W)])

  return k(users, items, user_factors, item_factors)


_BU_BLK = 1024


def _mm_body(u_ref, v_ref, o_ref):
  o_ref[...] = lax.dot_general(
      u_ref[...], v_ref[...],
      dimension_numbers=(((1,), (1,)), ((), ())),
      preferred_element_type=jnp.float32)


def _tc_matmul(u, v):
  return pl.pallas_call(
      _mm_body,
      grid=(B_U // _BU_BLK,),
      in_specs=[
          pl.BlockSpec((_BU_BLK, F), lambda i: (i, 0)),
          pl.BlockSpec((B_I, F), lambda i: (0, 0)),
      ],
      out_specs=pl.BlockSpec((_BU_BLK, B_I), lambda i: (i, 0)),
      out_shape=jax.ShapeDtypeStruct((B_U, B_I), jnp.float32),
      compiler_params=pltpu.CompilerParams(
          dimension_semantics=("arbitrary",)),
  )(u, v)


def kernel(users, items, user_factors, item_factors):
  users = users.astype(jnp.int32)
  items = items.astype(jnp.int32)
  u, v = _sc_gather(users, items, user_factors, item_factors)
  return _tc_matmul(u, v)
